# baseline (speedup 1.0000x reference)
import functools

import jax
import jax.numpy as jnp
from jax import lax
from jax.experimental import pallas as pl
from jax.experimental.pallas import tpu as pltpu
from jax.experimental.pallas import tpu_sc as plsc

F32 = jnp.float32
I32 = jnp.int32

_N_CORES = 2
_N_SUB = 16
_NW = _N_CORES * _N_SUB
_K = 128
_TC_BLK = 512
_DEG_W = 16


def _ceil_to(v, m):
    return (v + m - 1) // m * m


@functools.lru_cache(maxsize=None)
def _make_deg(n_pad, e_pad):
    nchunk = e_pad // (_NW * _K)
    rows_pt = n_pad // _N_SUB
    rseg = rows_pt // _K
    mesh = plsc.VectorSubcoreMesh(core_axis_name="c", subcore_axis_name="s")

    def body(dst_hbm, out_hbm, dstb, ones_v, zbuf, acc):
        c = lax.axis_index("c")
        s = lax.axis_index("s")
        wid = c * _N_SUB + s

        pltpu.sync_copy(dst_hbm.at[wid], dstb)

        def fill(i, _):
            ones_v[i, :] = jnp.ones((16,), F32)
            zbuf[i, :] = jnp.zeros((16,), F32)
            return 0

        lax.fori_loop(0, _K, fill, 0)

        def zinit(i, _):
            pltpu.sync_copy(zbuf, acc.at[pl.ds(s * rows_pt + i * _K, _K)])
            return 0

        lax.fori_loop(0, rseg, zinit, 0)
        plsc.subcore_barrier()

        def chunk(j, _):
            pltpu.sync_copy(ones_v, acc.at[dstb.at[j]], add=True)
            return 0

        lax.fori_loop(0, nchunk, chunk, 0)
        plsc.subcore_barrier()

        def wb(i, _):
            r0 = s * rows_pt + i * _K
            pltpu.sync_copy(acc.at[pl.ds(r0, _K)], zbuf)
            pltpu.sync_copy(zbuf, out_hbm.at[c, pl.ds(r0, _K)])
            return 0

        lax.fori_loop(0, rseg, wb, 0)

    return pl.kernel(
        body,
        out_type=jax.ShapeDtypeStruct((_N_CORES, n_pad, _DEG_W), F32),
        mesh=mesh,
        compiler_params=pltpu.CompilerParams(use_tc_tiling_on_sc=False),
        scratch_types=[
            pltpu.VMEM((nchunk, _K), I32),
            pltpu.VMEM((_K, _DEG_W), F32),
            pltpu.VMEM((_K, _DEG_W), F32),
            pltpu.VMEM_SHARED((n_pad, _DEG_W), F32),
        ],
    )


_NBUF = 2
_G = 16


@functools.lru_cache(maxsize=None)
def _make_agg(n_pad, e_pad, c_dim):
    nchunk = e_pad // (_NW * _K)
    ngrp = nchunk // _G
    rows_pt = n_pad // _N_SUB
    rseg = rows_pt // _K
    zero_row = n_pad - _K
    mesh = plsc.VectorSubcoreMesh(core_axis_name="c", subcore_axis_name="s")

    def body(y_hbm, src_hbm, dst_hbm, out_hbm,
             srcblk, dstblk, r0v, r1v, acc, s0, s1, sbs, sbd):
        c = lax.axis_index("c")
        s = lax.axis_index("s")
        wid = c * _N_SUB + s
        rows = (r0v, r1v)
        sems = (s0, s1)

        pltpu.async_copy(src_hbm.at[wid, pl.ds(0, _G)], srcblk.at[0], sbs)
        pltpu.async_copy(dst_hbm.at[wid, pl.ds(0, _G)], dstblk.at[0], sbd)

        def seed(i, _):
            dst0 = s * rows_pt + i * _K
            src0 = jnp.where(c == 0, dst0, zero_row)
            pltpu.sync_copy(y_hbm.at[pl.ds(src0, _K)], r0v)
            pltpu.sync_copy(r0v, acc.at[pl.ds(dst0, _K)])
            return 0

        lax.fori_loop(0, rseg, seed, 0)
        plsc.subcore_barrier()

        def grp(g, _):
            bb = lax.rem(g, 2)
            pltpu.make_async_copy(
                src_hbm.at[wid, pl.ds(0, _G)], srcblk.at[bb], sbs).wait()
            pltpu.make_async_copy(
                dst_hbm.at[wid, pl.ds(0, _G)], dstblk.at[bb], sbd).wait()

            @pl.when(g + 1 < ngrp)
            def _():
                nb = 1 - bb
                pltpu.async_copy(
                    src_hbm.at[wid, pl.ds((g + 1) * _G, _G)],
                    srcblk.at[nb], sbs)
                pltpu.async_copy(
                    dst_hbm.at[wid, pl.ds((g + 1) * _G, _G)],
                    dstblk.at[nb], sbd)

            for b in range(_NBUF):
                pltpu.async_copy(y_hbm.at[srcblk.at[bb, b]], rows[b], sems[b])
            for k in range(_G):
                b = k % _NBUF
                pltpu.make_async_copy(
                    y_hbm.at[srcblk.at[bb, 0]], rows[b], sems[b]).wait()
                pltpu.sync_copy(rows[b], acc.at[dstblk.at[bb, k]], add=True)
                if k + _NBUF < _G:
                    pltpu.async_copy(
                        y_hbm.at[srcblk.at[bb, k + _NBUF]], rows[b], sems[b])
            return 0

        lax.fori_loop(0, ngrp, grp, 0)
        plsc.subcore_barrier()

        def wb(i, _):
            r0 = s * rows_pt + i * _K
            pltpu.sync_copy(acc.at[pl.ds(r0, _K)], r0v)
            pltpu.sync_copy(r0v, out_hbm.at[c, pl.ds(r0, _K)])
            return 0

        lax.fori_loop(0, rseg, wb, 0)

    return pl.kernel(
        body,
        out_type=jax.ShapeDtypeStruct((_N_CORES, n_pad, c_dim), F32),
        mesh=mesh,
        compiler_params=pltpu.CompilerParams(use_tc_tiling_on_sc=False),
        scratch_types=[
            pltpu.VMEM((2, _G, _K), I32),
            pltpu.VMEM((2, _G, _K), I32),
            pltpu.VMEM((_K, c_dim), F32),
            pltpu.VMEM((_K, c_dim), F32),
            pltpu.VMEM_SHARED((n_pad, c_dim), F32),
            pltpu.SemaphoreType.DMA,
            pltpu.SemaphoreType.DMA,
            pltpu.SemaphoreType.DMA,
            pltpu.SemaphoreType.DMA,
        ],
    )


def _dinv_of(degp):
    deg = jnp.sum(degp, axis=0)[:, 0:1] + 1.0
    return lax.rsqrt(deg)


def _tc1_body(x_ref, w_ref, degp_ref, y_ref):
    dinv = _dinv_of(degp_ref[...])
    y_ref[...] = jnp.dot(x_ref[...], w_ref[...],
                         preferred_element_type=F32) * dinv


def _tc2_body(p_ref, degp_ref, b_ref, w_ref, y_ref, *, n_valid):
    dinv = _dinv_of(degp_ref[...])
    p = p_ref[...]
    h = jnp.maximum((p[0] + p[1]) * dinv + b_ref[...], 0.0)
    y = jnp.dot(h, w_ref[...], preferred_element_type=F32) * dinv
    ridx = (pl.program_id(0) * _TC_BLK
            + lax.broadcasted_iota(I32, (_TC_BLK, 1), 0))
    y_ref[...] = jnp.where(ridx < n_valid, y, 0.0)


def _tc3_body(q_ref, degp_ref, b_ref, wd_ref, bd_ref, logp_ref, emb_ref):
    dinv = _dinv_of(degp_ref[...])
    q = q_ref[...]
    h = (q[0] + q[1]) * dinv + b_ref[...]
    nrm = jnp.sqrt(jnp.sum(h * h, axis=1, keepdims=True))
    emb = h / (nrm + 1e-12)
    logits = jnp.dot(emb, wd_ref[...], preferred_element_type=F32) + bd_ref[...]
    m = jnp.max(logits, axis=1, keepdims=True)
    lse = jnp.log(jnp.sum(jnp.exp(logits - m), axis=1, keepdims=True)) + m
    logp_ref[...] = logits - lse
    emb_ref[...] = emb


def kernel(x, edge_index, W1, b1, W2, b2, Wd, bd):
    n, in_c = x.shape
    e = edge_index.shape[1]
    h1c = W1.shape[1]
    h2c = W2.shape[1]
    oc = Wd.shape[1]

    n_pad = _ceil_to(n + _K, _N_SUB * _K)
    e_pad = _ceil_to(e, _NW * _K * _G)
    nchunk = e_pad // (_NW * _K)
    grid = n_pad // _TC_BLK

    x_p = jnp.pad(x, ((0, n_pad - n), (0, 0)))
    pad_e = e_pad - e
    src_p = jnp.concatenate(
        [edge_index[0].astype(I32), jnp.full((pad_e,), n, I32)]
    ).reshape(_NW, nchunk, _K)
    dst_p = jnp.concatenate(
        [edge_index[1].astype(I32), jnp.full((pad_e,), n, I32)]
    ).reshape(_NW, nchunk, _K)

    degp = _make_deg(n_pad, e_pad)(dst_p)

    y1 = pl.pallas_call(
        _tc1_body,
        grid=grid,
        in_specs=[
            pl.BlockSpec((_TC_BLK, in_c), lambda i: (i, 0)),
            pl.BlockSpec((in_c, h1c), lambda i: (0, 0)),
            pl.BlockSpec((_N_CORES, _TC_BLK, _DEG_W), lambda i: (0, i, 0)),
        ],
        out_specs=pl.BlockSpec((_TC_BLK, h1c), lambda i: (i, 0)),
        out_shape=jax.ShapeDtypeStruct((n_pad, h1c), F32),
    )(x_p, W1, degp)

    p1 = _make_agg(n_pad, e_pad, h1c)(y1, src_p, dst_p)

    y2 = pl.pallas_call(
        functools.partial(_tc2_body, n_valid=n),
        grid=grid,
        in_specs=[
            pl.BlockSpec((_N_CORES, _TC_BLK, h1c), lambda i: (0, i, 0)),
            pl.BlockSpec((_N_CORES, _TC_BLK, _DEG_W), lambda i: (0, i, 0)),
            pl.BlockSpec((1, h1c), lambda i: (0, 0)),
            pl.BlockSpec((h1c, h2c), lambda i: (0, 0)),
        ],
        out_specs=pl.BlockSpec((_TC_BLK, h2c), lambda i: (i, 0)),
        out_shape=jax.ShapeDtypeStruct((n_pad, h2c), F32),
    )(p1, degp, b1.reshape(1, h1c), W2)

    p2 = _make_agg(n_pad, e_pad, h2c)(y2, src_p, dst_p)

    logp, emb = pl.pallas_call(
        _tc3_body,
        grid=grid,
        in_specs=[
            pl.BlockSpec((_N_CORES, _TC_BLK, h2c), lambda i: (0, i, 0)),
            pl.BlockSpec((_N_CORES, _TC_BLK, _DEG_W), lambda i: (0, i, 0)),
            pl.BlockSpec((1, h2c), lambda i: (0, 0)),
            pl.BlockSpec((h2c, oc), lambda i: (0, 0)),
            pl.BlockSpec((1, oc), lambda i: (0, 0)),
        ],
        out_specs=[
            pl.BlockSpec((_TC_BLK, oc), lambda i: (i, 0)),
            pl.BlockSpec((_TC_BLK, h2c), lambda i: (i, 0)),
        ],
        out_shape=[
            jax.ShapeDtypeStruct((n_pad, oc), F32),
            jax.ShapeDtypeStruct((n_pad, h2c), F32),
        ],
    )(p2, degp, b2.reshape(1, h2c), Wd, bd.reshape(1, oc))

    return logp[:n], emb[:n]


# speedup vs baseline: 2.5422x; 2.5422x over previous
import functools

import jax
import jax.numpy as jnp
from jax import lax
from jax.experimental import pallas as pl
from jax.experimental.pallas import tpu as pltpu
from jax.experimental.pallas import tpu_sc as plsc

F32 = jnp.float32
I32 = jnp.int32

_N_CORES = 2
_N_SUB = 16
_NW = _N_CORES * _N_SUB
_K = 128
_TC_BLK = 512
_DEG_W = 16


def _ceil_to(v, m):
    return (v + m - 1) // m * m


@functools.lru_cache(maxsize=None)
def _make_deg(n_pad, e_pad):
    nchunk = e_pad // (_NW * _K)
    rows_pt = n_pad // _N_SUB
    rseg = rows_pt // _K
    mesh = plsc.VectorSubcoreMesh(core_axis_name="c", subcore_axis_name="s")

    def body(dst_hbm, out_hbm, dstb, ones_v, zbuf, acc):
        c = lax.axis_index("c")
        s = lax.axis_index("s")
        wid = c * _N_SUB + s

        pltpu.sync_copy(dst_hbm.at[wid], dstb)

        def fill(i, _):
            ones_v[i, :] = jnp.ones((16,), F32)
            zbuf[i, :] = jnp.zeros((16,), F32)
            return 0

        lax.fori_loop(0, _K, fill, 0)

        def zinit(i, _):
            pltpu.sync_copy(zbuf, acc.at[pl.ds(s * rows_pt + i * _K, _K)])
            return 0

        lax.fori_loop(0, rseg, zinit, 0)
        plsc.subcore_barrier()

        def chunk(j, _):
            pltpu.sync_copy(ones_v, acc.at[dstb.at[j]], add=True)
            return 0

        lax.fori_loop(0, nchunk, chunk, 0)
        plsc.subcore_barrier()

        def wb(i, _):
            r0 = s * rows_pt + i * _K
            pltpu.sync_copy(acc.at[pl.ds(r0, _K)], zbuf)
            pltpu.sync_copy(zbuf, out_hbm.at[c, pl.ds(r0, _K)])
            return 0

        lax.fori_loop(0, rseg, wb, 0)

    return pl.kernel(
        body,
        out_type=jax.ShapeDtypeStruct((_N_CORES, n_pad, _DEG_W), F32),
        mesh=mesh,
        compiler_params=pltpu.CompilerParams(use_tc_tiling_on_sc=False),
        scratch_types=[
            pltpu.VMEM((nchunk, _K), I32),
            pltpu.VMEM((_K, _DEG_W), F32),
            pltpu.VMEM((_K, _DEG_W), F32),
            pltpu.VMEM_SHARED((n_pad, _DEG_W), F32),
        ],
    )


_NBUF = 2
_G = 16


@functools.lru_cache(maxsize=None)
def _make_agg(n_pad, e_pad, c_dim):
    nchunk = e_pad // (_NW * _K)
    ngrp = nchunk // _G
    rows_pt = n_pad // _N_SUB
    rseg = rows_pt // _K
    zero_row = n_pad - _K
    mesh = plsc.VectorSubcoreMesh(core_axis_name="c", subcore_axis_name="s")

    def body(y_hbm, src_hbm, dst_hbm, out_hbm,
             srcblk, dstblk, r0v, r1v, acc, s0, s1, sbs, sbd):
        c = lax.axis_index("c")
        s = lax.axis_index("s")
        wid = c * _N_SUB + s
        rows = (r0v, r1v)
        sems = (s0, s1)

        pltpu.async_copy(src_hbm.at[wid, pl.ds(0, _G)], srcblk.at[0], sbs)
        pltpu.async_copy(dst_hbm.at[wid, pl.ds(0, _G)], dstblk.at[0], sbd)

        def seed(i, _):
            dst0 = s * rows_pt + i * _K
            src0 = jnp.where(c == 0, dst0, zero_row)
            pltpu.sync_copy(y_hbm.at[pl.ds(src0, _K)], r0v)
            pltpu.sync_copy(r0v, acc.at[pl.ds(dst0, _K)])
            return 0

        lax.fori_loop(0, rseg, seed, 0)
        plsc.subcore_barrier()

        def grp(g, _):
            bb = lax.rem(g, 2)
            pltpu.make_async_copy(
                src_hbm.at[wid, pl.ds(0, _G)], srcblk.at[bb], sbs).wait()
            pltpu.make_async_copy(
                dst_hbm.at[wid, pl.ds(0, _G)], dstblk.at[bb], sbd).wait()

            @pl.when(g + 1 < ngrp)
            def _():
                nb = 1 - bb
                pltpu.async_copy(
                    src_hbm.at[wid, pl.ds((g + 1) * _G, _G)],
                    srcblk.at[nb], sbs)
                pltpu.async_copy(
                    dst_hbm.at[wid, pl.ds((g + 1) * _G, _G)],
                    dstblk.at[nb], sbd)

            for b in range(_NBUF):
                pltpu.async_copy(y_hbm.at[srcblk.at[bb, b]], rows[b], sems[b])
            for k in range(_G):
                b = k % _NBUF
                pltpu.make_async_copy(
                    y_hbm.at[srcblk.at[bb, 0]], rows[b], sems[b]).wait()
                pltpu.sync_copy(rows[b], acc.at[dstblk.at[bb, k]], add=True)
                if k + _NBUF < _G:
                    pltpu.async_copy(
                        y_hbm.at[srcblk.at[bb, k + _NBUF]], rows[b], sems[b])
            return 0

        lax.fori_loop(0, ngrp, grp, 0)
        plsc.subcore_barrier()

        def wb(i, _):
            r0 = s * rows_pt + i * _K
            pltpu.sync_copy(acc.at[pl.ds(r0, _K)], r0v)
            pltpu.sync_copy(r0v, out_hbm.at[c, pl.ds(r0, _K)])
            return 0

        lax.fori_loop(0, rseg, wb, 0)

    return pl.kernel(
        body,
        out_type=jax.ShapeDtypeStruct((_N_CORES, n_pad, c_dim), F32),
        mesh=mesh,
        compiler_params=pltpu.CompilerParams(use_tc_tiling_on_sc=False),
        scratch_types=[
            pltpu.VMEM((2, _G, _K), I32),
            pltpu.VMEM((2, _G, _K), I32),
            pltpu.VMEM((_K, c_dim), F32),
            pltpu.VMEM((_K, c_dim), F32),
            pltpu.VMEM_SHARED((n_pad, c_dim), F32),
            pltpu.SemaphoreType.DMA,
            pltpu.SemaphoreType.DMA,
            pltpu.SemaphoreType.DMA,
            pltpu.SemaphoreType.DMA,
        ],
    )


def _dinv_of(degp):
    deg = jnp.sum(degp, axis=0)[:, 0:1] + 1.0
    return lax.rsqrt(deg)


def _tc1_body(x_ref, w_ref, degp_ref, y_ref):
    dinv = _dinv_of(degp_ref[...])
    y_ref[...] = jnp.dot(x_ref[...], w_ref[...],
                         preferred_element_type=F32) * dinv


def _tc2_body(p_ref, degp_ref, b_ref, w_ref, y_ref, *, n_valid):
    dinv = _dinv_of(degp_ref[...])
    p = p_ref[...]
    h = jnp.maximum((p[0] + p[1]) * dinv + b_ref[...], 0.0)
    y = jnp.dot(h, w_ref[...], preferred_element_type=F32) * dinv
    ridx = (pl.program_id(0) * _TC_BLK
            + lax.broadcasted_iota(I32, (_TC_BLK, 1), 0))
    y_ref[...] = jnp.where(ridx < n_valid, y, 0.0)


def _tc3_body(q_ref, degp_ref, b_ref, wd_ref, bd_ref, logp_ref, emb_ref):
    dinv = _dinv_of(degp_ref[...])
    q = q_ref[...]
    h = (q[0] + q[1]) * dinv + b_ref[...]
    nrm = jnp.sqrt(jnp.sum(h * h, axis=1, keepdims=True))
    emb = h / (nrm + 1e-12)
    logits = jnp.dot(emb, wd_ref[...], preferred_element_type=F32) + bd_ref[...]
    m = jnp.max(logits, axis=1, keepdims=True)
    lse = jnp.log(jnp.sum(jnp.exp(logits - m), axis=1, keepdims=True)) + m
    logp_ref[...] = logits - lse
    emb_ref[...] = emb


def kernel(x, edge_index, W1, b1, W2, b2, Wd, bd):
    n, in_c = x.shape
    e = edge_index.shape[1]
    h1c = W1.shape[1]
    h2c = W2.shape[1]
    oc = Wd.shape[1]

    n_pad = _ceil_to(n + _K, _N_SUB * _K)
    e_pad = _ceil_to(e, _NW * _K * _G)
    nchunk = e_pad // (_NW * _K)
    grid = n_pad // _TC_BLK

    x_p = jnp.pad(x, ((0, n_pad - n), (0, 0)))
    pad_e = e_pad - e
    pad_idx = n + jnp.arange(pad_e, dtype=I32) % (n_pad - n)
    src_p = jnp.concatenate(
        [edge_index[0].astype(I32), pad_idx]).reshape(_NW, nchunk, _K)
    dst_p = jnp.concatenate(
        [edge_index[1].astype(I32), pad_idx]).reshape(_NW, nchunk, _K)

    degp = _make_deg(n_pad, e_pad)(dst_p)

    y1 = pl.pallas_call(
        _tc1_body,
        grid=grid,
        in_specs=[
            pl.BlockSpec((_TC_BLK, in_c), lambda i: (i, 0)),
            pl.BlockSpec((in_c, h1c), lambda i: (0, 0)),
            pl.BlockSpec((_N_CORES, _TC_BLK, _DEG_W), lambda i: (0, i, 0)),
        ],
        out_specs=pl.BlockSpec((_TC_BLK, h1c), lambda i: (i, 0)),
        out_shape=jax.ShapeDtypeStruct((n_pad, h1c), F32),
    )(x_p, W1, degp)

    p1 = _make_agg(n_pad, e_pad, h1c)(y1, src_p, dst_p)

    y2 = pl.pallas_call(
        functools.partial(_tc2_body, n_valid=n),
        grid=grid,
        in_specs=[
            pl.BlockSpec((_N_CORES, _TC_BLK, h1c), lambda i: (0, i, 0)),
            pl.BlockSpec((_N_CORES, _TC_BLK, _DEG_W), lambda i: (0, i, 0)),
            pl.BlockSpec((1, h1c), lambda i: (0, 0)),
            pl.BlockSpec((h1c, h2c), lambda i: (0, 0)),
        ],
        out_specs=pl.BlockSpec((_TC_BLK, h2c), lambda i: (i, 0)),
        out_shape=jax.ShapeDtypeStruct((n_pad, h2c), F32),
    )(p1, degp, b1.reshape(1, h1c), W2)

    p2 = _make_agg(n_pad, e_pad, h2c)(y2, src_p, dst_p)

    logp, emb = pl.pallas_call(
        _tc3_body,
        grid=grid,
        in_specs=[
            pl.BlockSpec((_N_CORES, _TC_BLK, h2c), lambda i: (0, i, 0)),
            pl.BlockSpec((_N_CORES, _TC_BLK, _DEG_W), lambda i: (0, i, 0)),
            pl.BlockSpec((1, h2c), lambda i: (0, 0)),
            pl.BlockSpec((h2c, oc), lambda i: (0, 0)),
            pl.BlockSpec((1, oc), lambda i: (0, 0)),
        ],
        out_specs=[
            pl.BlockSpec((_TC_BLK, oc), lambda i: (i, 0)),
            pl.BlockSpec((_TC_BLK, h2c), lambda i: (i, 0)),
        ],
        out_shape=[
            jax.ShapeDtypeStruct((n_pad, oc), F32),
            jax.ShapeDtypeStruct((n_pad, h2c), F32),
        ],
    )(p2, degp, b2.reshape(1, h2c), Wd, bd.reshape(1, oc))

    return logp[:n], emb[:n]


# speedup vs baseline: 2.6255x; 1.0328x over previous
import functools

import jax
import jax.numpy as jnp
from jax import lax
from jax.experimental import pallas as pl
from jax.experimental.pallas import tpu as pltpu
from jax.experimental.pallas import tpu_sc as plsc

F32 = jnp.float32
I32 = jnp.int32

_N_CORES = 2
_N_SUB = 16
_NW = _N_CORES * _N_SUB
_K = 128
_TC_BLK = 512
_DEG_W = 16


def _ceil_to(v, m):
    return (v + m - 1) // m * m


@functools.lru_cache(maxsize=None)
def _make_deg(n_pad, e_pad):
    nchunk = e_pad // (_NW * _K)
    rows_pt = n_pad // _N_SUB
    rseg = rows_pt // _K
    mesh = plsc.VectorSubcoreMesh(core_axis_name="c", subcore_axis_name="s")

    def body(dst_hbm, zeros_hbm, out_hbm, dstb, ones_v, acc):
        c = lax.axis_index("c")
        s = lax.axis_index("s")
        wid = c * _N_SUB + s

        pltpu.sync_copy(dst_hbm.at[wid], dstb)

        def fill(i, _):
            ones_v[i, :] = jnp.ones((16,), F32)
            return 0

        lax.fori_loop(0, _K, fill, 0)

        tile_rows = pl.ds(s * rows_pt, rows_pt)
        pltpu.sync_copy(zeros_hbm, acc.at[tile_rows])
        plsc.subcore_barrier()

        def chunk(j, _):
            pltpu.sync_copy(ones_v, acc.at[dstb.at[j]], add=True)
            return 0

        lax.fori_loop(0, nchunk, chunk, 0)
        plsc.subcore_barrier()

        pltpu.sync_copy(acc.at[tile_rows], out_hbm.at[c, tile_rows])

    return pl.kernel(
        body,
        out_type=jax.ShapeDtypeStruct((_N_CORES, n_pad, _DEG_W), F32),
        mesh=mesh,
        compiler_params=pltpu.CompilerParams(use_tc_tiling_on_sc=False),
        scratch_types=[
            pltpu.VMEM((nchunk, _K), I32),
            pltpu.VMEM((_K, _DEG_W), F32),
            pltpu.VMEM_SHARED((n_pad, _DEG_W), F32),
        ],
    )


_NBUF = 2
_G = 16


@functools.lru_cache(maxsize=None)
def _make_agg(n_pad, e_pad, c_dim):
    nchunk = e_pad // (_NW * _K)
    ngrp = nchunk // _G
    rows_pt = n_pad // _N_SUB
    rseg = rows_pt // _K
    zero_row = n_pad - _K
    mesh = plsc.VectorSubcoreMesh(core_axis_name="c", subcore_axis_name="s")

    def body(y_hbm, zeros_hbm, src_hbm, dst_hbm, out_hbm,
             srcblk, dstblk, r0v, r1v, acc, s0, s1, sbs, sbd):
        c = lax.axis_index("c")
        s = lax.axis_index("s")
        wid = c * _N_SUB + s
        rows = (r0v, r1v)
        sems = (s0, s1)

        pltpu.async_copy(src_hbm.at[wid, pl.ds(0, _G)], srcblk.at[0], sbs)
        pltpu.async_copy(dst_hbm.at[wid, pl.ds(0, _G)], dstblk.at[0], sbd)

        tile_rows = pl.ds(s * rows_pt, rows_pt)

        @pl.when(c == 0)
        def _():
            pltpu.sync_copy(y_hbm.at[tile_rows], acc.at[tile_rows])

        @pl.when(c == 1)
        def _():
            pltpu.sync_copy(zeros_hbm, acc.at[tile_rows])

        plsc.subcore_barrier()

        def grp(g, _):
            bb = lax.rem(g, 2)
            pltpu.make_async_copy(
                src_hbm.at[wid, pl.ds(0, _G)], srcblk.at[bb], sbs).wait()
            pltpu.make_async_copy(
                dst_hbm.at[wid, pl.ds(0, _G)], dstblk.at[bb], sbd).wait()

            @pl.when(g + 1 < ngrp)
            def _():
                nb = 1 - bb
                pltpu.async_copy(
                    src_hbm.at[wid, pl.ds((g + 1) * _G, _G)],
                    srcblk.at[nb], sbs)
                pltpu.async_copy(
                    dst_hbm.at[wid, pl.ds((g + 1) * _G, _G)],
                    dstblk.at[nb], sbd)

            for b in range(_NBUF):
                pltpu.async_copy(y_hbm.at[srcblk.at[bb, b]], rows[b], sems[b])
            for k in range(_G):
                b = k % _NBUF
                pltpu.make_async_copy(
                    y_hbm.at[srcblk.at[bb, 0]], rows[b], sems[b]).wait()
                pltpu.sync_copy(rows[b], acc.at[dstblk.at[bb, k]], add=True)
                if k + _NBUF < _G:
                    pltpu.async_copy(
                        y_hbm.at[srcblk.at[bb, k + _NBUF]], rows[b], sems[b])
            return 0

        lax.fori_loop(0, ngrp, grp, 0)
        plsc.subcore_barrier()

        pltpu.sync_copy(acc.at[tile_rows], out_hbm.at[c, tile_rows])

    return pl.kernel(
        body,
        out_type=jax.ShapeDtypeStruct((_N_CORES, n_pad, c_dim), F32),
        mesh=mesh,
        compiler_params=pltpu.CompilerParams(use_tc_tiling_on_sc=False),
        scratch_types=[
            pltpu.VMEM((2, _G, _K), I32),
            pltpu.VMEM((2, _G, _K), I32),
            pltpu.VMEM((_K, c_dim), F32),
            pltpu.VMEM((_K, c_dim), F32),
            pltpu.VMEM_SHARED((n_pad, c_dim), F32),
            pltpu.SemaphoreType.DMA,
            pltpu.SemaphoreType.DMA,
            pltpu.SemaphoreType.DMA,
            pltpu.SemaphoreType.DMA,
        ],
    )


def _dinv_of(degp):
    deg = jnp.sum(degp, axis=0)[:, 0:1] + 1.0
    return lax.rsqrt(deg)


def _tc1_body(x_ref, w_ref, degp_ref, y_ref, *, n_valid):
    dinv = _dinv_of(degp_ref[...])
    y = jnp.dot(x_ref[...], w_ref[...], preferred_element_type=F32) * dinv
    ridx = (pl.program_id(0) * _TC_BLK
            + lax.broadcasted_iota(I32, (_TC_BLK, 1), 0))
    y_ref[...] = jnp.where(ridx < n_valid, y, 0.0)


def _tc2_body(p_ref, degp_ref, b_ref, w_ref, y_ref, *, n_valid):
    dinv = _dinv_of(degp_ref[...])
    p = p_ref[...]
    h = jnp.maximum((p[0] + p[1]) * dinv + b_ref[...], 0.0)
    y = jnp.dot(h, w_ref[...], preferred_element_type=F32) * dinv
    ridx = (pl.program_id(0) * _TC_BLK
            + lax.broadcasted_iota(I32, (_TC_BLK, 1), 0))
    y_ref[...] = jnp.where(ridx < n_valid, y, 0.0)


def _tc3_body(q_ref, degp_ref, b_ref, wd_ref, bd_ref, logp_ref, emb_ref):
    dinv = _dinv_of(degp_ref[...])
    q = q_ref[...]
    h = (q[0] + q[1]) * dinv + b_ref[...]
    nrm = jnp.sqrt(jnp.sum(h * h, axis=1, keepdims=True))
    emb = h / (nrm + 1e-12)
    logits = jnp.dot(emb, wd_ref[...], preferred_element_type=F32) + bd_ref[...]
    m = jnp.max(logits, axis=1, keepdims=True)
    lse = jnp.log(jnp.sum(jnp.exp(logits - m), axis=1, keepdims=True)) + m
    logp_ref[...] = logits - lse
    emb_ref[...] = emb


def kernel(x, edge_index, W1, b1, W2, b2, Wd, bd):
    n, in_c = x.shape
    e = edge_index.shape[1]
    h1c = W1.shape[1]
    h2c = W2.shape[1]
    oc = Wd.shape[1]

    n_pad = _ceil_to(n + _K, _N_SUB * _K)
    e_pad = _ceil_to(e, _NW * _K * _G)
    nchunk = e_pad // (_NW * _K)
    grid = n_pad // _TC_BLK

    rows_pt = n_pad // _N_SUB
    pad_e = e_pad - e
    pad_idx = n + jnp.arange(pad_e, dtype=I32) % (n_pad - n)
    src_p = jnp.concatenate(
        [edge_index[0].astype(I32), pad_idx]).reshape(_NW, nchunk, _K)
    dst_p = jnp.concatenate(
        [edge_index[1].astype(I32), pad_idx]).reshape(_NW, nchunk, _K)

    degp = _make_deg(n_pad, e_pad)(
        dst_p, jnp.zeros((rows_pt, _DEG_W), F32))

    y1 = pl.pallas_call(
        functools.partial(_tc1_body, n_valid=n),
        grid=grid,
        in_specs=[
            pl.BlockSpec((_TC_BLK, in_c), lambda i: (i, 0)),
            pl.BlockSpec((in_c, h1c), lambda i: (0, 0)),
            pl.BlockSpec((_N_CORES, _TC_BLK, _DEG_W), lambda i: (0, i, 0)),
        ],
        out_specs=pl.BlockSpec((_TC_BLK, h1c), lambda i: (i, 0)),
        out_shape=jax.ShapeDtypeStruct((n_pad, h1c), F32),
    )(x, W1, degp)

    p1 = _make_agg(n_pad, e_pad, h1c)(
        y1, jnp.zeros((rows_pt, h1c), F32), src_p, dst_p)

    y2 = pl.pallas_call(
        functools.partial(_tc2_body, n_valid=n),
        grid=grid,
        in_specs=[
            pl.BlockSpec((_N_CORES, _TC_BLK, h1c), lambda i: (0, i, 0)),
            pl.BlockSpec((_N_CORES, _TC_BLK, _DEG_W), lambda i: (0, i, 0)),
            pl.BlockSpec((1, h1c), lambda i: (0, 0)),
            pl.BlockSpec((h1c, h2c), lambda i: (0, 0)),
        ],
        out_specs=pl.BlockSpec((_TC_BLK, h2c), lambda i: (i, 0)),
        out_shape=jax.ShapeDtypeStruct((n_pad, h2c), F32),
    )(p1, degp, b1.reshape(1, h1c), W2)

    p2 = _make_agg(n_pad, e_pad, h2c)(
        y2, jnp.zeros((rows_pt, h2c), F32), src_p, dst_p)

    logp, emb = pl.pallas_call(
        _tc3_body,
        grid=grid,
        in_specs=[
            pl.BlockSpec((_N_CORES, _TC_BLK, h2c), lambda i: (0, i, 0)),
            pl.BlockSpec((_N_CORES, _TC_BLK, _DEG_W), lambda i: (0, i, 0)),
            pl.BlockSpec((1, h2c), lambda i: (0, 0)),
            pl.BlockSpec((h2c, oc), lambda i: (0, 0)),
            pl.BlockSpec((1, oc), lambda i: (0, 0)),
        ],
        out_specs=[
            pl.BlockSpec((_TC_BLK, oc), lambda i: (i, 0)),
            pl.BlockSpec((_TC_BLK, h2c), lambda i: (i, 0)),
        ],
        out_shape=[
            jax.ShapeDtypeStruct((n, oc), F32),
            jax.ShapeDtypeStruct((n, h2c), F32),
        ],
    )(p2, degp, b2.reshape(1, h2c), Wd, bd.reshape(1, oc))

    return logp, emb


# speedup vs baseline: 2.8738x; 1.0946x over previous
import functools

import jax
import jax.numpy as jnp
from jax import lax
from jax.experimental import pallas as pl
from jax.experimental.pallas import tpu as pltpu
from jax.experimental.pallas import tpu_sc as plsc

F32 = jnp.float32
I32 = jnp.int32

_N_CORES = 2
_N_SUB = 16
_NW = _N_CORES * _N_SUB
_K = 128
_TC_BLK = 1024
_DEG_W = 16


def _ceil_to(v, m):
    return (v + m - 1) // m * m


def _load_tile_indices(idx_hbm, pad_hbm, buf, wid, nchunk, vchunk, sem):
    base = wid * nchunk

    def issue(j, _):
        g = base + j

        @pl.when(g < vchunk)
        def _():
            pltpu.async_copy(idx_hbm.at[0, pl.ds(g * _K, _K)],
                             buf.at[pl.ds(j * _K, _K)], sem)

        @pl.when(g >= vchunk)
        def _():
            pltpu.async_copy(pad_hbm.at[g - vchunk],
                             buf.at[pl.ds(j * _K, _K)], sem)
        return 0

    lax.fori_loop(0, nchunk, issue, 0)

    def drain(j, _):
        pltpu.make_async_copy(
            idx_hbm.at[0, pl.ds(0, _K)], buf.at[pl.ds(0, _K)],
            sem).wait()
        return 0

    lax.fori_loop(0, nchunk, drain, 0)


@functools.lru_cache(maxsize=None)
def _make_deg(n_pad, e_pad, e):
    nchunk = e_pad // (_NW * _K)
    vchunk = e // _K
    rows_pt = n_pad // _N_SUB
    mesh = plsc.VectorSubcoreMesh(core_axis_name="c", subcore_axis_name="s")

    def body(dst_hbm, pad_hbm, zeros_hbm, out_hbm, dstb, ones_v, acc, sem):
        c = lax.axis_index("c")
        s = lax.axis_index("s")
        wid = c * _N_SUB + s

        _load_tile_indices(dst_hbm, pad_hbm, dstb, wid, nchunk, vchunk, sem)

        def fill(i, _):
            ones_v[i, :] = jnp.ones((16,), F32)
            return 0

        lax.fori_loop(0, _K, fill, 0)

        tile_rows = pl.ds(s * rows_pt, rows_pt)
        pltpu.sync_copy(zeros_hbm, acc.at[tile_rows])
        plsc.subcore_barrier()

        def chunk(j, _):
            pltpu.sync_copy(ones_v, acc.at[dstb.at[pl.ds(j * _K, _K)]],
                            add=True)
            return 0

        lax.fori_loop(0, nchunk, chunk, 0)
        plsc.subcore_barrier()

        pltpu.sync_copy(acc.at[tile_rows], out_hbm.at[c, tile_rows])

    return pl.kernel(
        body,
        out_type=jax.ShapeDtypeStruct((_N_CORES, n_pad, _DEG_W), F32),
        mesh=mesh,
        compiler_params=pltpu.CompilerParams(use_tc_tiling_on_sc=False),
        scratch_types=[
            pltpu.VMEM((nchunk * _K,), I32),
            pltpu.VMEM((_K, _DEG_W), F32),
            pltpu.VMEM_SHARED((n_pad, _DEG_W), F32),
            pltpu.SemaphoreType.DMA,
        ],
    )


_NBUF = 2


@functools.lru_cache(maxsize=None)
def _make_agg(n, n_pad, e_pad, e, c_dim):
    nchunk = e_pad // (_NW * _K)
    vchunk = e // _K
    rows_pt = n_pad // _N_SUB
    mesh = plsc.VectorSubcoreMesh(core_axis_name="c", subcore_axis_name="s")

    n_zero = n_pad - n
    ring = 6

    def body(y_hbm, src_hbm, dst_hbm, pad_hbm, out_hbm,
             srcb, dstb, r0v, r1v, acc, s_src, s_dst, s0, s1):
        c = lax.axis_index("c")
        s = lax.axis_index("s")
        wid = c * _N_SUB + s
        rows = (r0v, r1v)
        sems = (s0, s1)
        base = wid * nchunk

        def issue_idx(j):
            g = base + j
            slot = lax.rem(j, ring) if not isinstance(j, int) else j % ring

            @pl.when(g < vchunk)
            def _():
                pltpu.async_copy(src_hbm.at[0, pl.ds(g * _K, _K)],
                                 srcb.at[slot], s_src)
                pltpu.async_copy(dst_hbm.at[0, pl.ds(g * _K, _K)],
                                 dstb.at[slot], s_dst)

            @pl.when(g >= vchunk)
            def _():
                pltpu.async_copy(pad_hbm.at[g - vchunk], srcb.at[slot],
                                 s_src)
                pltpu.async_copy(pad_hbm.at[g - vchunk], dstb.at[slot],
                                 s_dst)

        def wait_one(buf, sem):
            pltpu.make_async_copy(
                src_hbm.at[0, pl.ds(0, _K)], buf.at[0], sem).wait()

        for j in range(min(ring - 1, nchunk)):
            issue_idx(j)
        for b in range(min(_NBUF, nchunk)):
            wait_one(srcb, s_src)
            pltpu.async_copy(y_hbm.at[srcb.at[b]], rows[b], sems[b])

        tile_rows = pl.ds(s * rows_pt, rows_pt)

        @pl.when(c == 0)
        def _():
            pltpu.sync_copy(y_hbm.at[tile_rows], acc.at[tile_rows])

        @pl.when(c == 1)
        def _():
            off = 0
            while off < rows_pt:
                sz = min(n_zero, rows_pt - off)
                pltpu.sync_copy(
                    y_hbm.at[pl.ds(n, sz)],
                    acc.at[pl.ds(s * rows_pt + off, sz)])
                off += sz

        plsc.subcore_barrier()

        def step(j, b, refill=True):
            wait_one(dstb, s_dst)
            pltpu.make_async_copy(
                y_hbm.at[srcb.at[0]], rows[b], sems[b]).wait()
            pltpu.sync_copy(rows[b], acc.at[dstb.at[lax.rem(j, ring)]],
                            add=True)
            if refill:
                @pl.when(j + _NBUF < nchunk)
                def _():
                    wait_one(srcb, s_src)
                    pltpu.async_copy(
                        y_hbm.at[srcb.at[lax.rem(j + _NBUF, ring)]],
                        rows[b], sems[b])

                @pl.when(j + ring - 1 < nchunk)
                def _():
                    issue_idx(j + ring - 1)

        def pair(g, _):
            step(2 * g, 0)
            step(2 * g + 1, 1)
            return 0

        lax.fori_loop(0, nchunk // 2, pair, 0)
        if nchunk % 2:
            step(nchunk - 1, 0, refill=False)
        plsc.subcore_barrier()

        pltpu.sync_copy(acc.at[tile_rows], out_hbm.at[c, tile_rows])

    return pl.kernel(
        body,
        out_type=jax.ShapeDtypeStruct((_N_CORES, n_pad, c_dim), F32),
        mesh=mesh,
        compiler_params=pltpu.CompilerParams(use_tc_tiling_on_sc=False),
        scratch_types=[
            pltpu.VMEM((ring, _K), I32),
            pltpu.VMEM((ring, _K), I32),
            pltpu.VMEM((_K, c_dim), F32),
            pltpu.VMEM((_K, c_dim), F32),
            pltpu.VMEM_SHARED((n_pad, c_dim), F32),
            pltpu.SemaphoreType.DMA,
            pltpu.SemaphoreType.DMA,
            pltpu.SemaphoreType.DMA,
            pltpu.SemaphoreType.DMA,
        ],
    )


def _dinv_of(degp):
    deg = jnp.sum(degp, axis=0)[:, 0:1] + 1.0
    return lax.rsqrt(deg)


def _tc1_body(x_ref, w_ref, degp_ref, y_ref, *, n_valid):
    dinv = _dinv_of(degp_ref[...])
    y = jnp.dot(x_ref[...], w_ref[...], preferred_element_type=F32) * dinv
    ridx = (pl.program_id(0) * _TC_BLK
            + lax.broadcasted_iota(I32, (_TC_BLK, 1), 0))
    y_ref[...] = jnp.where(ridx < n_valid, y, 0.0)


def _tc2_body(p_ref, degp_ref, b_ref, w_ref, y_ref, *, n_valid):
    dinv = _dinv_of(degp_ref[...])
    p = p_ref[...]
    h = jnp.maximum((p[0] + p[1]) * dinv + b_ref[...], 0.0)
    y = jnp.dot(h, w_ref[...], preferred_element_type=F32) * dinv
    ridx = (pl.program_id(0) * _TC_BLK
            + lax.broadcasted_iota(I32, (_TC_BLK, 1), 0))
    y_ref[...] = jnp.where(ridx < n_valid, y, 0.0)


def _tc3_body(q_ref, degp_ref, b_ref, wd_ref, bd_ref, logp_ref, emb_ref):
    dinv = _dinv_of(degp_ref[...])
    q = q_ref[...]
    h = (q[0] + q[1]) * dinv + b_ref[...]
    nrm = jnp.sqrt(jnp.sum(h * h, axis=1, keepdims=True))
    emb = h / (nrm + 1e-12)
    logits = jnp.dot(emb, wd_ref[...], preferred_element_type=F32) + bd_ref[...]
    m = jnp.max(logits, axis=1, keepdims=True)
    lse = jnp.log(jnp.sum(jnp.exp(logits - m), axis=1, keepdims=True)) + m
    logp_ref[...] = logits - lse
    emb_ref[...] = emb


def kernel(x, edge_index, W1, b1, W2, b2, Wd, bd):
    n, in_c = x.shape
    e = edge_index.shape[1]
    h1c = W1.shape[1]
    h2c = W2.shape[1]
    oc = Wd.shape[1]

    n_pad = _ceil_to(n + _K, _N_SUB * _K)
    e_ck = _ceil_to(e, _K)
    e_pad = _ceil_to(e_ck, _NW * _K)
    nchunk = e_pad // (_NW * _K)
    grid = n_pad // _TC_BLK

    rows_pt = n_pad // _N_SUB

    src = edge_index[0].astype(I32)
    dst = edge_index[1].astype(I32)
    if e_ck != e:
        tail = n + jnp.arange(e_ck - e, dtype=I32) % (n_pad - n)
        src = jnp.concatenate([src, tail])
        dst = jnp.concatenate([dst, tail])
    src = src.reshape(1, e_ck)
    dst = dst.reshape(1, e_ck)
    n_pad_ck = max(e_pad - e_ck, _K)
    pad_tab = (n + jnp.arange(n_pad_ck, dtype=I32) % (n_pad - n)
               ).reshape(-1, _K)

    degp = _make_deg(n_pad, e_pad, e_ck)(
        dst, pad_tab, jnp.zeros((rows_pt, _DEG_W), F32))

    y1 = pl.pallas_call(
        functools.partial(_tc1_body, n_valid=n),
        grid=grid,
        in_specs=[
            pl.BlockSpec((_TC_BLK, in_c), lambda i: (i, 0)),
            pl.BlockSpec((in_c, h1c), lambda i: (0, 0)),
            pl.BlockSpec((_N_CORES, _TC_BLK, _DEG_W), lambda i: (0, i, 0)),
        ],
        out_specs=pl.BlockSpec((_TC_BLK, h1c), lambda i: (i, 0)),
        out_shape=jax.ShapeDtypeStruct((n_pad, h1c), F32),
    )(x, W1, degp)

    p1 = _make_agg(n, n_pad, e_pad, e_ck, h1c)(y1, src, dst, pad_tab)

    y2 = pl.pallas_call(
        functools.partial(_tc2_body, n_valid=n),
        grid=grid,
        in_specs=[
            pl.BlockSpec((_N_CORES, _TC_BLK, h1c), lambda i: (0, i, 0)),
            pl.BlockSpec((_N_CORES, _TC_BLK, _DEG_W), lambda i: (0, i, 0)),
            pl.BlockSpec((1, h1c), lambda i: (0, 0)),
            pl.BlockSpec((h1c, h2c), lambda i: (0, 0)),
        ],
        out_specs=pl.BlockSpec((_TC_BLK, h2c), lambda i: (i, 0)),
        out_shape=jax.ShapeDtypeStruct((n_pad, h2c), F32),
    )(p1, degp, b1.reshape(1, h1c), W2)

    p2 = _make_agg(n, n_pad, e_pad, e_ck, h2c)(y2, src, dst, pad_tab)

    logp, emb = pl.pallas_call(
        _tc3_body,
        grid=grid,
        in_specs=[
            pl.BlockSpec((_N_CORES, _TC_BLK, h2c), lambda i: (0, i, 0)),
            pl.BlockSpec((_N_CORES, _TC_BLK, _DEG_W), lambda i: (0, i, 0)),
            pl.BlockSpec((1, h2c), lambda i: (0, 0)),
            pl.BlockSpec((h2c, oc), lambda i: (0, 0)),
            pl.BlockSpec((1, oc), lambda i: (0, 0)),
        ],
        out_specs=[
            pl.BlockSpec((_TC_BLK, oc), lambda i: (i, 0)),
            pl.BlockSpec((_TC_BLK, h2c), lambda i: (i, 0)),
        ],
        out_shape=[
            jax.ShapeDtypeStruct((n, oc), F32),
            jax.ShapeDtypeStruct((n, h2c), F32),
        ],
    )(p2, degp, b2.reshape(1, h2c), Wd, bd.reshape(1, oc))

    return logp, emb
